# baseline (speedup 1.0000x reference)
import jax
import jax.numpy as jnp
from jax import lax
from jax.experimental import pallas as pl
from jax.experimental.pallas import tpu as pltpu
from jax.experimental.pallas import tpu_sc as plsc

N = 2048
E = 8192
H = 128
K = 64
G = 32

NC = 2
NS = 16
NW = NC * NS
BPW = E // NW
NSEG = N // NS


def _sc_mesh():
    return plsc.VectorSubcoreMesh(core_axis_name="c", subcore_axis_name="s")


L2 = 128
RPW = BPW // L2


def _sc_gather(h, idx2d):
    def body(h_hbm, idx_hbm, out_hbm, idx_v, rows_v, sem, sem2):
        wid = lax.axis_index("s") * NC + lax.axis_index("c")
        row0 = wid * RPW
        pltpu.sync_copy(idx_hbm.at[pl.ds(row0, RPW)], idx_v)
        cps = [pltpu.async_copy(h_hbm.at[idx_v.at[j]], rows_v.at[j], sem)
               for j in range(RPW)]
        for cp in cps:
            cp.wait()
        wps = [pltpu.async_copy(rows_v.at[j],
                                out_hbm.at[pl.ds((row0 + j) * L2, L2)], sem2)
               for j in range(RPW)]
        for wp in wps:
            wp.wait()

    return pl.kernel(
        body,
        out_type=jax.ShapeDtypeStruct((E, H), jnp.float32),
        mesh=_sc_mesh(),
        scratch_types=[
            pltpu.VMEM((RPW, L2), jnp.int32),
            pltpu.VMEM((RPW, L2, H), jnp.float32),
            pltpu.SemaphoreType.DMA,
            pltpu.SemaphoreType.DMA,
        ],
    )(h, idx2d)


def _sc_scatter(msg, idx2d, zeros):
    ne = msg.shape[0]
    bpw = ne // NW
    rpw = bpw // L2

    def body(msg_hbm, idx_hbm, z_hbm, out_hbm, idx_v, rows_v, acc_sh, sem):
        c = lax.axis_index("c")
        s = lax.axis_index("s")
        wid = s * NC + c
        row0 = wid * rpw
        zcp = pltpu.async_copy(z_hbm.at[pl.ds(s * NSEG, NSEG)],
                               acc_sh.at[pl.ds(s * NSEG, NSEG)], sem)
        pltpu.sync_copy(idx_hbm.at[pl.ds(row0, rpw)], idx_v)
        pltpu.sync_copy(msg_hbm.at[pl.ds(row0 * L2, rpw * L2)],
                        rows_v)
        zcp.wait()
        plsc.subcore_barrier()
        for j in range(rpw):
            pltpu.sync_copy(rows_v.at[pl.ds(j * L2, L2)],
                            acc_sh.at[idx_v.at[j]], add=True)
        plsc.subcore_barrier()
        pltpu.sync_copy(acc_sh.at[pl.ds(s * NSEG, NSEG)],
                        out_hbm.at[c, pl.ds(s * NSEG, NSEG)])

    return pl.kernel(
        body,
        out_type=jax.ShapeDtypeStruct((NC, N, H), jnp.float32),
        mesh=_sc_mesh(),
        scratch_types=[
            pltpu.VMEM((rpw, L2), jnp.int32),
            pltpu.VMEM((bpw, H), jnp.float32),
            pltpu.VMEM_SHARED((N, H), jnp.float32),
            pltpu.SemaphoreType.DMA,
        ],
    )(msg, idx2d, zeros)


def _tc_msg(ea2, w1, b1r, hs, w2, b2r):
    TE = 256

    def body(ea_ref, w1_ref, b1_ref, hs_ref, w2_ref, bb_ref, out_ref):
        hm = jnp.maximum(ea_ref[...] * w1_ref[...] + b1_ref[...], 0.0)
        w_tile = (jnp.dot(hm.astype(jnp.bfloat16), w2_ref[...],
                          preferred_element_type=jnp.float32)
                  + bb_ref[...]).astype(jnp.bfloat16).reshape(TE, H, H)
        out_ref[...] = lax.dot_general(
            hs_ref[...].astype(jnp.bfloat16), w_tile,
            (((1,), (1,)), ((0,), (0,))),
            preferred_element_type=jnp.float32)

    ne = hs.shape[0]
    return pl.pallas_call(
        body,
        grid=(ne // TE,),
        in_specs=[
            pl.BlockSpec((TE, 1), lambda i: (i, 0)),
            pl.BlockSpec((1, K), lambda i: (0, 0)),
            pl.BlockSpec((1, K), lambda i: (0, 0)),
            pl.BlockSpec((TE, H), lambda i: (i, 0)),
            pl.BlockSpec((K, H * H), lambda i: (0, 0)),
            pl.BlockSpec((1, H * H), lambda i: (0, 0)),
        ],
        out_specs=pl.BlockSpec((TE, H), lambda i: (i, 0)),
        out_shape=jax.ShapeDtypeStruct((ne, H), jnp.float32),
    )(ea2, w1, b1r, hs, w2, b2r)


def _tc_inv(dst2d):
    def body(d_ref, out_ref):
        nids = lax.broadcasted_iota(jnp.int32, (N, 1), 0)

        def step(r, acc):
            row = d_ref[pl.ds(r, 1), :]
            return acc + jnp.sum((row == nids).astype(jnp.float32),
                                 axis=1, keepdims=True)

        acc = lax.fori_loop(0, E // H, step, jnp.zeros((N, 1), jnp.float32))
        out_ref[...] = 1.0 / jnp.maximum(acc, 1.0)

    return pl.pallas_call(
        body,
        out_shape=jax.ShapeDtypeStruct((N, 1), jnp.float32),
    )(dst2d)


def _tc_update(s2a, s2b, hprev, root, cbr, inv, resid):
    TN = 256

    def body(sa_ref, sb_ref, h_ref, r_ref, cb_ref, inv_ref, out_ref):
        mean = ((sa_ref[0] + sa_ref[1]) + (sb_ref[0] + sb_ref[1])) \
            * inv_ref[...]
        o = (mean
             + jnp.dot(h_ref[...], r_ref[...],
                       preferred_element_type=jnp.float32)
             + cb_ref[...])
        if resid:
            o = o + h_ref[...]
        out_ref[...] = jnp.maximum(o, 0.0)

    return pl.pallas_call(
        body,
        grid=(N // TN,),
        in_specs=[
            pl.BlockSpec((NC, TN, H), lambda i: (0, i, 0)),
            pl.BlockSpec((NC, TN, H), lambda i: (0, i, 0)),
            pl.BlockSpec((TN, H), lambda i: (i, 0)),
            pl.BlockSpec((H, H), lambda i: (0, 0)),
            pl.BlockSpec((1, H), lambda i: (0, 0)),
            pl.BlockSpec((TN, 1), lambda i: (i, 0)),
        ],
        out_specs=pl.BlockSpec((TN, H), lambda i: (i, 0)),
        out_shape=jax.ShapeDtypeStruct((N, H), jnp.float32),
    )(s2a, s2b, hprev, root, cbr, inv)


def _tc_final(s2a, s2b, hprev, root, cbr, inv, b2d, gw1, gb1r, gw2t, gb2r,
              l1w, l1br, l2w, l2br, l3w, l3br, lwt, lbr):
    def body(sa_ref, sb_ref, h_ref, r_ref, cb_ref, inv_ref, b_ref,
             gw1_ref, gb1_ref, gw2_ref, gb2_ref,
             l1w_ref, l1b_ref, l2w_ref, l2b_ref, l3w_ref, l3b_ref,
             lw_ref, lb_ref, out_ref):
        hp = h_ref[...]
        x3v = jnp.maximum(
            ((sa_ref[0] + sa_ref[1]) + (sb_ref[0] + sb_ref[1]))
            * inv_ref[...]
            + jnp.dot(hp, r_ref[...], preferred_element_type=jnp.float32)
            + cb_ref[...] + hp, 0.0)
        gl = jnp.maximum(
            jnp.dot(x3v, gw1_ref[...], preferred_element_type=jnp.float32)
            + gb1_ref[...], 0.0)
        g = (jnp.dot(gl, gw2_ref[...], preferred_element_type=jnp.float32)
             + gb2_ref[...])
        oh = b_ref[...] == lax.broadcasted_iota(jnp.int32, (1, G), 1)
        ohf = oh.astype(jnp.float32)
        neg = jnp.float32(-jnp.inf)
        gmax = jnp.max(jnp.where(oh, g, neg), axis=0, keepdims=True)
        gmax = jnp.where(jnp.isfinite(gmax), gmax, 0.0)
        gmax_n = jnp.sum(ohf * gmax, axis=1, keepdims=True)
        ex = jnp.exp(g - gmax_n)
        den = jnp.sum(ohf * ex, axis=0, keepdims=True)
        den_n = jnp.sum(ohf * den, axis=1, keepdims=True)
        wgt = ex / (den_n + 1e-16)
        wx = wgt * x3v
        bf = jnp.bfloat16
        hi = wx.astype(bf)
        lo = (wx - hi.astype(jnp.float32)).astype(bf)
        lo2 = (wx - hi.astype(jnp.float32) - lo.astype(jnp.float32)).astype(bf)
        cdims = (((0,), (0,)), ((), ()))
        pooled = (lax.dot_general(ohf, hi.astype(jnp.float32), cdims,
                                  preferred_element_type=jnp.float32)
                  + lax.dot_general(ohf, lo.astype(jnp.float32), cdims,
                                    preferred_element_type=jnp.float32)
                  + lax.dot_general(ohf, lo2.astype(jnp.float32), cdims,
                                    preferred_element_type=jnp.float32))
        hh = jnp.maximum(
            jnp.dot(pooled, l1w_ref[...], preferred_element_type=jnp.float32)
            + l1b_ref[...], 0.0)
        hh = jnp.maximum(
            jnp.dot(hh, l2w_ref[...], preferred_element_type=jnp.float32)
            + l2b_ref[...], 0.0)
        hh = jnp.maximum(
            jnp.dot(hh, l3w_ref[...], preferred_element_type=jnp.float32)
            + l3b_ref[...], 0.0)
        out_ref[...] = (jnp.dot(hh, lw_ref[...],
                                preferred_element_type=jnp.float32)
                        + lb_ref[...])

    return pl.pallas_call(
        body,
        out_shape=jax.ShapeDtypeStruct((G, 1), jnp.float32),
    )(s2a, s2b, hprev, root, cbr, inv, b2d, gw1, gb1r, gw2t, gb2r,
      l1w, l1br, l2w, l2br, l3w, l3br, lwt, lbr)


def kernel(x, edge_index, edge_attr, batch_index,
           m1w1, m1b1, m1w2, m1b2,
           m2w1, m2b1, m2w2, m2b2,
           m3w1, m3b1, m3w2, m3b2,
           root1, cb1, root2, cb2, root3, cb3,
           gw1, gb1, gw2, gb2,
           l1w, l1b, l2w, l2b, l3w, l3b, lw, lb):
    src = edge_index[0].reshape(E // L2, L2)
    dst = edge_index[1].reshape(E // L2, L2)
    ea2 = edge_attr[:, None]
    dst2d = edge_index[1].reshape(E // H, H)
    zeros = jnp.zeros((N, H), jnp.float32)

    inv = _tc_inv(dst2d)

    layers = [
        (m1w1, m1b1, m1w2, m1b2, root1, cb1, False),
        (m2w1, m2b1, m2w2, m2b2, root2, cb2, True),
        (m3w1, m3b1, m3w2, m3b2, root3, cb3, True),
    ]
    EH = E // 2

    def half_msgs(h, w1, b1, w2b, b2r):
        hs = _sc_gather(h, src)
        s2h = []
        for hf in range(2):
            msg = _tc_msg(ea2[hf * EH:(hf + 1) * EH], w1, b1,
                          hs[hf * EH:(hf + 1) * EH], w2b, b2r)
            s2h.append(_sc_scatter(
                msg, dst[hf * (EH // L2):(hf + 1) * (EH // L2)], zeros))
        return s2h

    h = x
    for w1, b1, w2, b2, root, cb, resid in layers[:2]:
        s2a, s2b = half_msgs(h, w1, b1.reshape(1, K),
                             w2.astype(jnp.bfloat16), b2.reshape(1, H * H))
        h = _tc_update(s2a, s2b, h, root, cb.reshape(1, H), inv, resid)

    w1, b1, w2, b2, root, cb, _ = layers[2]
    s2a, s2b = half_msgs(h, w1, b1.reshape(1, K),
                         w2.astype(jnp.bfloat16), b2.reshape(1, H * H))

    return _tc_final(s2a, s2b, h, root, cb.reshape(1, H), inv,
                     batch_index[:, None], gw1, gb1.reshape(1, K),
                     gw2, gb2.reshape(1, 1),
                     l1w, l1b.reshape(1, H), l2w, l2b.reshape(1, K),
                     l3w, l3b.reshape(1, 16), lw,
                     lb.reshape(1, 1))


# speedup vs baseline: 1.0435x; 1.0435x over previous
import jax
import jax.numpy as jnp
from jax import lax
from jax.experimental import pallas as pl
from jax.experimental.pallas import tpu as pltpu
from jax.experimental.pallas import tpu_sc as plsc

N = 2048
E = 8192
H = 128
K = 64
G = 32

NC = 2
NS = 16
NW = NC * NS
BPW = E // NW
NSEG = N // NS


def _sc_mesh():
    return plsc.VectorSubcoreMesh(core_axis_name="c", subcore_axis_name="s")


L2 = 128
RPW = BPW // L2


def _sc_gather(h, idx2d):
    def body(h_hbm, idx_hbm, out_hbm, idx_v, rows_v, sem, sem2):
        wid = lax.axis_index("s") * NC + lax.axis_index("c")
        row0 = wid * RPW
        pltpu.sync_copy(idx_hbm.at[pl.ds(row0, RPW)], idx_v)
        cps = [pltpu.async_copy(h_hbm.at[idx_v.at[j]], rows_v.at[j], sem)
               for j in range(RPW)]
        for cp in cps:
            cp.wait()
        wps = [pltpu.async_copy(rows_v.at[j],
                                out_hbm.at[pl.ds((row0 + j) * L2, L2)], sem2)
               for j in range(RPW)]
        for wp in wps:
            wp.wait()

    return pl.kernel(
        body,
        out_type=jax.ShapeDtypeStruct((E, H), jnp.float32),
        mesh=_sc_mesh(),
        scratch_types=[
            pltpu.VMEM((RPW, L2), jnp.int32),
            pltpu.VMEM((RPW, L2, H), jnp.float32),
            pltpu.SemaphoreType.DMA,
            pltpu.SemaphoreType.DMA,
        ],
    )(h, idx2d)


def _sc_scatter(msg, idx2d, zeros):
    ne = msg.shape[0]
    bpw = ne // NW
    rpw = bpw // L2

    def body(msg_hbm, idx_hbm, z_hbm, out_hbm, idx_v, rows_v, acc_sh, sem):
        c = lax.axis_index("c")
        s = lax.axis_index("s")
        wid = s * NC + c
        row0 = wid * rpw
        zcp = pltpu.async_copy(z_hbm.at[pl.ds(s * NSEG, NSEG)],
                               acc_sh.at[pl.ds(s * NSEG, NSEG)], sem)
        pltpu.sync_copy(idx_hbm.at[pl.ds(row0, rpw)], idx_v)
        pltpu.sync_copy(msg_hbm.at[pl.ds(row0 * L2, rpw * L2)],
                        rows_v)
        zcp.wait()
        plsc.subcore_barrier()
        for j in range(rpw):
            pltpu.sync_copy(rows_v.at[pl.ds(j * L2, L2)],
                            acc_sh.at[idx_v.at[j]], add=True)
        plsc.subcore_barrier()
        pltpu.sync_copy(acc_sh.at[pl.ds(s * NSEG, NSEG)],
                        out_hbm.at[c, pl.ds(s * NSEG, NSEG)])

    return pl.kernel(
        body,
        out_type=jax.ShapeDtypeStruct((NC, N, H), jnp.float32),
        mesh=_sc_mesh(),
        scratch_types=[
            pltpu.VMEM((rpw, L2), jnp.int32),
            pltpu.VMEM((bpw, H), jnp.float32),
            pltpu.VMEM_SHARED((N, H), jnp.float32),
            pltpu.SemaphoreType.DMA,
        ],
    )(msg, idx2d, zeros)


def _tc_msg(ea2, w1, b1r, hs, w2, b2r):
    TE = 256

    def body(ea_ref, w1_ref, b1_ref, hs_ref, w2_ref, bb_ref, out_ref):
        hm = jnp.maximum(ea_ref[...] * w1_ref[...] + b1_ref[...], 0.0)
        w_tile = (jnp.dot(hm.astype(jnp.bfloat16), w2_ref[...],
                          preferred_element_type=jnp.float32)
                  + bb_ref[...]).astype(jnp.bfloat16).reshape(TE, H, H)
        out_ref[...] = lax.dot_general(
            hs_ref[...].astype(jnp.bfloat16), w_tile,
            (((1,), (1,)), ((0,), (0,))),
            preferred_element_type=jnp.float32)

    ne = hs.shape[0]
    return pl.pallas_call(
        body,
        grid=(ne // TE,),
        in_specs=[
            pl.BlockSpec((TE, 1), lambda i: (i, 0)),
            pl.BlockSpec((1, K), lambda i: (0, 0)),
            pl.BlockSpec((1, K), lambda i: (0, 0)),
            pl.BlockSpec((TE, H), lambda i: (i, 0)),
            pl.BlockSpec((K, H * H), lambda i: (0, 0)),
            pl.BlockSpec((1, H * H), lambda i: (0, 0)),
        ],
        out_specs=pl.BlockSpec((TE, H), lambda i: (i, 0)),
        out_shape=jax.ShapeDtypeStruct((ne, H), jnp.float32),
    )(ea2, w1, b1r, hs, w2, b2r)


def _tc_inv(dst2d):
    def body(d_ref, out_ref):
        nids = lax.broadcasted_iota(jnp.int32, (N, 1), 0)

        def step(r, acc):
            row = d_ref[pl.ds(r, 1), :]
            return acc + jnp.sum((row == nids).astype(jnp.float32),
                                 axis=1, keepdims=True)

        acc = lax.fori_loop(0, E // H, step, jnp.zeros((N, 1), jnp.float32))
        out_ref[...] = 1.0 / jnp.maximum(acc, 1.0)

    return pl.pallas_call(
        body,
        out_shape=jax.ShapeDtypeStruct((N, 1), jnp.float32),
    )(dst2d)


def _tc_update(s2, hprev, root, cbr, inv, resid):
    TN = 256

    def body(s_ref, h_ref, r_ref, cb_ref, inv_ref, out_ref):
        mean = (s_ref[0] + s_ref[1]) * inv_ref[...]
        o = (mean
             + jnp.dot(h_ref[...], r_ref[...],
                       preferred_element_type=jnp.float32)
             + cb_ref[...])
        if resid:
            o = o + h_ref[...]
        out_ref[...] = jnp.maximum(o, 0.0)

    return pl.pallas_call(
        body,
        grid=(N // TN,),
        in_specs=[
            pl.BlockSpec((NC, TN, H), lambda i: (0, i, 0)),
            pl.BlockSpec((TN, H), lambda i: (i, 0)),
            pl.BlockSpec((H, H), lambda i: (0, 0)),
            pl.BlockSpec((1, H), lambda i: (0, 0)),
            pl.BlockSpec((TN, 1), lambda i: (i, 0)),
        ],
        out_specs=pl.BlockSpec((TN, H), lambda i: (i, 0)),
        out_shape=jax.ShapeDtypeStruct((N, H), jnp.float32),
    )(s2, hprev, root, cbr, inv)


def _tc_final(s2, hprev, root, cbr, inv, b2d, gw1, gb1r, gw2t, gb2r,
              l1w, l1br, l2w, l2br, l3w, l3br, lwt, lbr):
    def body(s_ref, h_ref, r_ref, cb_ref, inv_ref, b_ref,
             gw1_ref, gb1_ref, gw2_ref, gb2_ref,
             l1w_ref, l1b_ref, l2w_ref, l2b_ref, l3w_ref, l3b_ref,
             lw_ref, lb_ref, out_ref):
        hp = h_ref[...]
        x3v = jnp.maximum(
            (s_ref[0] + s_ref[1]) * inv_ref[...]
            + jnp.dot(hp, r_ref[...], preferred_element_type=jnp.float32)
            + cb_ref[...] + hp, 0.0)
        gl = jnp.maximum(
            jnp.dot(x3v, gw1_ref[...], preferred_element_type=jnp.float32)
            + gb1_ref[...], 0.0)
        g = (jnp.dot(gl, gw2_ref[...], preferred_element_type=jnp.float32)
             + gb2_ref[...])
        oh = b_ref[...] == lax.broadcasted_iota(jnp.int32, (1, G), 1)
        ohf = oh.astype(jnp.float32)
        neg = jnp.float32(-jnp.inf)
        gmax = jnp.max(jnp.where(oh, g, neg), axis=0, keepdims=True)
        gmax = jnp.where(jnp.isfinite(gmax), gmax, 0.0)
        gmax_n = jnp.sum(ohf * gmax, axis=1, keepdims=True)
        ex = jnp.exp(g - gmax_n)
        den = jnp.sum(ohf * ex, axis=0, keepdims=True)
        den_n = jnp.sum(ohf * den, axis=1, keepdims=True)
        wgt = ex / (den_n + 1e-16)
        wx = wgt * x3v
        bf = jnp.bfloat16
        hi = wx.astype(bf)
        lo = (wx - hi.astype(jnp.float32)).astype(bf)
        lo2 = (wx - hi.astype(jnp.float32) - lo.astype(jnp.float32)).astype(bf)
        cdims = (((0,), (0,)), ((), ()))
        pooled = (lax.dot_general(ohf, hi.astype(jnp.float32), cdims,
                                  preferred_element_type=jnp.float32)
                  + lax.dot_general(ohf, lo.astype(jnp.float32), cdims,
                                    preferred_element_type=jnp.float32)
                  + lax.dot_general(ohf, lo2.astype(jnp.float32), cdims,
                                    preferred_element_type=jnp.float32))
        hh = jnp.maximum(
            jnp.dot(pooled, l1w_ref[...], preferred_element_type=jnp.float32)
            + l1b_ref[...], 0.0)
        hh = jnp.maximum(
            jnp.dot(hh, l2w_ref[...], preferred_element_type=jnp.float32)
            + l2b_ref[...], 0.0)
        hh = jnp.maximum(
            jnp.dot(hh, l3w_ref[...], preferred_element_type=jnp.float32)
            + l3b_ref[...], 0.0)
        out_ref[...] = (jnp.dot(hh, lw_ref[...],
                                preferred_element_type=jnp.float32)
                        + lb_ref[...])

    return pl.pallas_call(
        body,
        out_shape=jax.ShapeDtypeStruct((G, 1), jnp.float32),
    )(s2, hprev, root, cbr, inv, b2d, gw1, gb1r, gw2t, gb2r,
      l1w, l1br, l2w, l2br, l3w, l3br, lwt, lbr)


def kernel(x, edge_index, edge_attr, batch_index,
           m1w1, m1b1, m1w2, m1b2,
           m2w1, m2b1, m2w2, m2b2,
           m3w1, m3b1, m3w2, m3b2,
           root1, cb1, root2, cb2, root3, cb3,
           gw1, gb1, gw2, gb2,
           l1w, l1b, l2w, l2b, l3w, l3b, lw, lb):
    src = edge_index[0].reshape(E // L2, L2)
    dst = edge_index[1].reshape(E // L2, L2)
    ea2 = edge_attr[:, None]
    dst2d = edge_index[1].reshape(E // H, H)
    zeros = jnp.zeros((N, H), jnp.float32)

    inv = _tc_inv(dst2d)

    layers = [
        (m1w1, m1b1, m1w2, m1b2, root1, cb1, False),
        (m2w1, m2b1, m2w2, m2b2, root2, cb2, True),
        (m3w1, m3b1, m3w2, m3b2, root3, cb3, True),
    ]
    h = x
    for w1, b1, w2, b2, root, cb, resid in layers[:2]:
        hs = _sc_gather(h, src)
        msg = _tc_msg(ea2, w1, b1.reshape(1, K), hs,
                      w2.astype(jnp.bfloat16), b2.reshape(1, H * H))
        s2 = _sc_scatter(msg, dst, zeros)
        h = _tc_update(s2, h, root, cb.reshape(1, H), inv, resid)

    w1, b1, w2, b2, root, cb, _ = layers[2]
    hs = _sc_gather(h, src)
    msg = _tc_msg(ea2, w1, b1.reshape(1, K), hs,
                  w2.astype(jnp.bfloat16), b2.reshape(1, H * H))
    s2 = _sc_scatter(msg, dst, zeros)

    return _tc_final(s2, h, root, cb.reshape(1, H), inv,
                     batch_index[:, None], gw1, gb1.reshape(1, K),
                     gw2, gb2.reshape(1, 1),
                     l1w, l1b.reshape(1, H), l2w, l2b.reshape(1, K),
                     l3w, l3b.reshape(1, 16), lw,
                     lb.reshape(1, 1))
